# Initial kernel scaffold; baseline (speedup 1.0000x reference)
import jax
import jax.numpy as jnp
from jax.experimental import pallas as pl


def kernel(x, Wr, Ws, bs):
    raise NotImplementedError("write your pallas kernel here")


# speedup vs baseline: 1.6878x; 1.6878x over previous
import functools

import jax
import jax.numpy as jnp
from jax import lax
from jax.experimental import pallas as pl
from jax.experimental.pallas import tpu as pltpu
from jax.experimental.pallas import tpu_sc as plsc

E = 8
D = 1024
O = 1024
B = 2048
K = 32
BM = 128
NBLK = B // BM + E
S = NBLK * BM
RB = 256


def _window_sum(x):
    acc = x[:, 0:K]
    for p in range(1, D // K):
        acc = acc + x[:, p * K:(p + 1) * K]
    return acc


def _router_body(x_ref, wr_ref, sel0_ref, rank_ref, dpsum_ref,
                 cnt_ref, run_cnt):
    i = pl.program_id(0)

    @pl.when(i == 0)
    def _():
        run_cnt[...] = jnp.zeros_like(run_cnt)
        dpsum_ref[...] = jnp.zeros_like(dpsum_ref)
        cnt_ref[...] = jnp.zeros_like(cnt_ref)

    x = x_ref[...]
    logits = lax.dot_general(_window_sum(x), wr_ref[...],
                             (((1,), (1,)), ((), ())),
                             preferred_element_type=jnp.float32)
    m = jnp.max(logits, axis=1, keepdims=True)
    eexp = jnp.exp(logits - m)
    ssum = jnp.sum(eexp, axis=1, keepdims=True)
    iota_e = lax.broadcasted_iota(jnp.int32, (RB, E), 1)
    idx2 = jnp.min(jnp.where(logits == m, iota_e, E), axis=1, keepdims=True)
    onehot = (iota_e == idx2).astype(jnp.float32)
    sel0_ref[...] = onehot
    r_i = lax.broadcasted_iota(jnp.int32, (RB, RB), 0)
    c_i = lax.broadcasted_iota(jnp.int32, (RB, RB), 1)
    tri = (c_i < r_i).astype(jnp.float32)
    ranks = lax.dot_general(tri, onehot, (((1,), (0,)), ((), ())),
                            preferred_element_type=jnp.float32)
    rank_intra = jnp.sum(ranks * onehot, axis=1, keepdims=True)
    prev = run_cnt[...]
    rank_ref[...] = (rank_intra
                     + jnp.sum(onehot * prev, axis=1, keepdims=True)
                     ).astype(jnp.int32)
    blk_cnt = jnp.sum(onehot, axis=0, keepdims=True)
    run_cnt[...] = prev + blk_cnt
    dpsum_ref[...] += jnp.sum(eexp / ssum, axis=0, keepdims=True)
    cnt_ref[...] += blk_cnt


def _router_call(x, Wr):
    return pl.pallas_call(
        _router_body,
        grid=(B // RB,),
        in_specs=[pl.BlockSpec((RB, D), lambda i: (i, 0)),
                  pl.BlockSpec((E, K), lambda i: (0, 0))],
        out_specs=[pl.BlockSpec((RB, E), lambda i: (i, 0)),
                   pl.BlockSpec((RB, 1), lambda i: (i, 0)),
                   pl.BlockSpec((1, E), lambda i: (0, 0)),
                   pl.BlockSpec((1, E), lambda i: (0, 0))],
        out_shape=[jax.ShapeDtypeStruct((B, E), jnp.float32),
                   jax.ShapeDtypeStruct((B, 1), jnp.int32),
                   jax.ShapeDtypeStruct((1, E), jnp.float32),
                   jax.ShapeDtypeStruct((1, E), jnp.float32)],
        scratch_shapes=[pltpu.VMEM((1, E), jnp.float32)],
    )(x, Wr)


def _meta_body(sel0_ref, rank_ref, dpsum_ref, cnt_ref, pos_ref, bexp_ref,
               bval_ref, loss_ref):
    counts = cnt_ref[...]
    ci = counts.astype(jnp.int32)
    pcount = ((ci + (BM - 1)) // BM) * BM
    e_r = lax.broadcasted_iota(jnp.int32, (E, E), 0)
    e_c = lax.broadcasted_iota(jnp.int32, (E, E), 1)
    upper = (e_r < e_c).astype(jnp.float32)
    pstart = lax.dot_general(pcount.astype(jnp.float32), upper,
                             (((1,), (0,)), ((), ())),
                             preferred_element_type=jnp.float32)
    onehot = sel0_ref[...]
    ps_tok = jnp.sum(onehot * pstart, axis=1, keepdims=True)
    pos_ref[...] = rank_ref[...] + ps_tok.astype(jnp.int32)
    bstart = BM * lax.broadcasted_iota(jnp.int32, (NBLK, E), 0)
    pstart_b = jnp.broadcast_to(pstart.astype(jnp.int32), (NBLK, E))
    pcount_b = jnp.broadcast_to(pcount, (NBLK, E))
    cond = (bstart >= pstart_b) & (bstart < pstart_b + pcount_b)
    e_iota = lax.broadcasted_iota(jnp.int32, (NBLK, E), 1)
    bexp_ref[...] = jnp.sum(jnp.where(cond, e_iota, 0), axis=1, keepdims=True)
    bval_ref[...] = jnp.sum(cond.astype(jnp.int32), axis=1, keepdims=True)
    dp = dpsum_ref[...] / B
    dens = counts / B
    loss_ref[...] = jnp.sum(dp * dens, keepdims=True).reshape(1, 1) * E


def _meta_call(sel0, rank, dpsum, cnt):
    return pl.pallas_call(
        _meta_body,
        out_shape=[jax.ShapeDtypeStruct((B, 1), jnp.int32),
                   jax.ShapeDtypeStruct((NBLK, 1), jnp.int32),
                   jax.ShapeDtypeStruct((NBLK, 1), jnp.int32),
                   jax.ShapeDtypeStruct((1, 1), jnp.float32)],
    )(sel0, rank, dpsum, cnt)


@functools.lru_cache(maxsize=None)
def _sc_builders():
    info = plsc.get_sparse_core_info()
    nw = info.num_cores * info.num_subcores
    bpw = B // nw
    mesh = plsc.VectorSubcoreMesh(core_axis_name="c", subcore_axis_name="s")

    def _wid():
        return lax.axis_index("s") * info.num_cores + lax.axis_index("c")

    @functools.partial(
        pl.kernel,
        out_type=jax.ShapeDtypeStruct((S, D), jnp.float32),
        mesh=mesh,
        scratch_types=[pltpu.VMEM((bpw,), jnp.int32),
                       pltpu.VMEM((bpw, D), jnp.float32),
                       pltpu.SemaphoreType.DMA],
    )
    def scatter_k(x_hbm, pos_hbm, xs_hbm, idx_v, rows_v, sem):
        base = _wid() * bpw
        pltpu.sync_copy(pos_hbm.at[pl.ds(base, bpw)], idx_v)
        pltpu.sync_copy(x_hbm.at[pl.ds(base, bpw)], rows_v)
        pltpu.async_copy(rows_v, xs_hbm.at[idx_v], sem).wait()

    @functools.partial(
        pl.kernel,
        out_type=jax.ShapeDtypeStruct((B, O), jnp.float32),
        mesh=mesh,
        scratch_types=[pltpu.VMEM((bpw,), jnp.int32),
                       pltpu.VMEM((bpw, O), jnp.float32),
                       pltpu.SemaphoreType.DMA],
    )
    def gather_k(outs_hbm, pos_hbm, out_hbm, idx_v, rows_v, sem):
        base = _wid() * bpw
        pltpu.sync_copy(pos_hbm.at[pl.ds(base, bpw)], idx_v)
        pltpu.async_copy(outs_hbm.at[idx_v], rows_v, sem).wait()
        pltpu.sync_copy(rows_v, out_hbm.at[pl.ds(base, bpw)])

    return scatter_k, gather_k


def _mm_body(bexp_ref, bval_ref, xs_ref, ws_ref, bs_ref, wr_ref, out_ref):
    j = pl.program_id(0)

    @pl.when(bval_ref[j] != 0)
    def _():
        xsb = xs_ref[...]
        lg = lax.dot_general(_window_sum(xsb), wr_ref[...],
                             (((1,), (1,)), ((), ())),
                             preferred_element_type=jnp.float32)
        gate = 1.0 / jnp.sum(
            jnp.exp(lg - jnp.max(lg, axis=1, keepdims=True)),
            axis=1, keepdims=True)
        h = lax.dot_general(xsb, ws_ref[0], (((1,), (1,)), ((), ())),
                            preferred_element_type=jnp.float32)
        h = h + bs_ref[0]
        y = h * h * h * gate
        m = jnp.max(y, axis=1, keepdims=True)
        p = jnp.exp(y - m)
        out_ref[...] = p / jnp.sum(p, axis=1, keepdims=True)


def _mm_call(bexp, bval, xs, Ws, bs, Wr):
    grid_spec = pltpu.PrefetchScalarGridSpec(
        num_scalar_prefetch=2,
        grid=(NBLK,),
        in_specs=[pl.BlockSpec((BM, D), lambda j, be, bv: (j, 0)),
                  pl.BlockSpec((1, O, D), lambda j, be, bv: (be[j], 0, 0)),
                  pl.BlockSpec((1, 1, O), lambda j, be, bv: (be[j], 0, 0)),
                  pl.BlockSpec((E, K), lambda j, be, bv: (0, 0))],
        out_specs=pl.BlockSpec((BM, O), lambda j, be, bv: (j, 0)),
    )
    return pl.pallas_call(
        _mm_body,
        grid_spec=grid_spec,
        out_shape=jax.ShapeDtypeStruct((S, O), jnp.float32),
    )(bexp, bval, xs, Ws, bs.reshape(E, 1, O), Wr)


def kernel(x, Wr, Ws, bs):
    sel0, rank, dpsum, cnt = _router_call(x, Wr)
    pos, bexp, bval, loss = _meta_call(sel0, rank, dpsum, cnt)
    pos1 = pos.reshape(B)
    scatter_k, gather_k = _sc_builders()
    xs = scatter_k(x, pos1)
    outs = _mm_call(bexp.reshape(NBLK), bval.reshape(NBLK), xs, Ws, bs, Wr)
    output = gather_k(outs, pos1)
    return output, sel0, loss.reshape(())
